# baseline (speedup 1.0000x reference)
import jax
import jax.numpy as jnp
from jax.experimental import pallas as pl
from jax.experimental.pallas import tpu as pltpu

_CHUNK_T = 256
_N_BUF = 8


def _gate_rows(x, wt, bias):
    scores = jnp.dot(x, wt, preferred_element_type=jnp.float32) + bias
    e = scores.shape[-1]
    lane = jax.lax.broadcasted_iota(jnp.int32, scores.shape, 1)

    m1 = jnp.max(scores, axis=-1, keepdims=True)
    idx1 = jnp.min(jnp.where(scores == m1, lane, e), axis=-1, keepdims=True)
    first = lane == idx1
    s2 = jnp.where(first, -jnp.inf, scores)
    m2 = jnp.max(s2, axis=-1, keepdims=True)
    idx2 = jnp.min(jnp.where(s2 == m2, lane, e), axis=-1, keepdims=True)
    mask = first | (lane == idx2)

    p = jnp.exp(scores - m1)
    z = jnp.sum(p, axis=-1, keepdims=True)
    soft = p / z
    w = jnp.where(mask, soft, jnp.float32(0.0))
    s = jnp.sum(w, axis=-1, keepdims=True)
    return w / (s + jnp.float32(1e-8))


def _gate_kernel(x_hbm, wt_ref, b_ref, o_ref, xbuf, sem):
    n_chunks = x_hbm.shape[0] // _CHUNK_T
    wt = wt_ref[...]
    bias = b_ref[...]

    def start_copy(c, slot):
        pltpu.make_async_copy(
            x_hbm.at[pl.ds(c * _CHUNK_T, _CHUNK_T), :],
            xbuf.at[slot],
            sem.at[slot],
        ).start()

    for i in range(_N_BUF):
        start_copy(i, i)

    def body(c, carry):
        slot = jax.lax.rem(c, _N_BUF)
        pltpu.make_async_copy(
            x_hbm.at[pl.ds(c * _CHUNK_T, _CHUNK_T), :],
            xbuf.at[slot],
            sem.at[slot],
        ).wait()
        w = _gate_rows(xbuf[slot], wt, bias)
        o_ref[pl.ds(c * _CHUNK_T, _CHUNK_T), :] = w
        nxt = c + _N_BUF

        @pl.when(nxt < n_chunks)
        def _():
            start_copy(nxt, slot)

        return carry

    jax.lax.fori_loop(0, n_chunks, body, 0)


@jax.jit
def kernel(x, W, b):
    n_tokens, d_model = x.shape
    n_experts = W.shape[0]
    wt = W.T
    b2 = b.reshape(1, n_experts)
    return pl.pallas_call(
        _gate_kernel,
        in_specs=[
            pl.BlockSpec(memory_space=pl.ANY),
            pl.BlockSpec((d_model, n_experts), lambda: (0, 0)),
            pl.BlockSpec((1, n_experts), lambda: (0, 0)),
        ],
        out_specs=pl.BlockSpec((n_tokens, n_experts), lambda: (0, 0)),
        out_shape=jax.ShapeDtypeStruct((n_tokens, n_experts), jnp.float32),
        scratch_shapes=[
            pltpu.VMEM((_N_BUF, _CHUNK_T, d_model), jnp.float32),
            pltpu.SemaphoreType.DMA((_N_BUF,)),
        ],
    )(x, wt, b2)


# speedup vs baseline: 1.4241x; 1.4241x over previous
import functools

import jax
import jax.numpy as jnp
from jax.experimental import pallas as pl
from jax.experimental.pallas import tpu as pltpu

_BLOCK_T = 2048
_N_SPLIT = 8
_SUB_T = _BLOCK_T // _N_SPLIT


def _gate_rows(x, wt, bias):
    scores = jnp.dot(x, wt, preferred_element_type=jnp.float32) + bias
    e = scores.shape[-1]
    lane = jax.lax.broadcasted_iota(jnp.int32, scores.shape, 1)

    m1 = jnp.max(scores, axis=-1, keepdims=True)
    idx1 = jnp.min(jnp.where(scores == m1, lane, e), axis=-1, keepdims=True)
    first = lane == idx1
    s2 = jnp.where(first, -jnp.inf, scores)
    m2 = jnp.max(s2, axis=-1, keepdims=True)
    idx2 = jnp.min(jnp.where(s2 == m2, lane, e), axis=-1, keepdims=True)
    mask = first | (lane == idx2)

    p = jnp.exp(scores - m1)
    z = jnp.sum(p, axis=-1, keepdims=True)
    soft = p / z
    w = jnp.where(mask, soft, jnp.float32(0.0))
    s = jnp.sum(w, axis=-1, keepdims=True)
    return w / (s + jnp.float32(1e-8))


def _gate_kernel(*refs):
    x_refs = refs[:_N_SPLIT]
    wt_ref, b_ref, o_ref = refs[_N_SPLIT:]
    wt = wt_ref[...]
    bias = b_ref[...]
    for j in range(_N_SPLIT):
        w = _gate_rows(x_refs[j][...], wt, bias)
        o_ref[j * _SUB_T:(j + 1) * _SUB_T, :] = w


@jax.jit
def kernel(x, W, b):
    n_tokens, d_model = x.shape
    n_experts = W.shape[0]
    wt = W.T
    b2 = b.reshape(1, n_experts)
    grid = (n_tokens // _BLOCK_T,)
    x_specs = [
        pl.BlockSpec((_SUB_T, d_model),
                     functools.partial(lambda i, j: (_N_SPLIT * i + j, 0), j=j))
        for j in range(_N_SPLIT)
    ]
    return pl.pallas_call(
        _gate_kernel,
        grid=grid,
        in_specs=x_specs + [
            pl.BlockSpec((d_model, n_experts), lambda i: (0, 0)),
            pl.BlockSpec((1, n_experts), lambda i: (0, 0)),
        ],
        out_specs=pl.BlockSpec((_BLOCK_T, n_experts), lambda i: (i, 0)),
        out_shape=jax.ShapeDtypeStruct((n_tokens, n_experts), jnp.float32),
        compiler_params=pltpu.CompilerParams(
            dimension_semantics=("arbitrary",),
        ),
    )(*([x] * _N_SPLIT), wt, b2)


# speedup vs baseline: 1.4701x; 1.0323x over previous
import functools

import jax
import jax.numpy as jnp
from jax.experimental import pallas as pl
from jax.experimental.pallas import tpu as pltpu

_BLOCK_T = 2048
_N_SPLIT = 4
_SUB_T = _BLOCK_T // _N_SPLIT


def _gate_rows(x, wt, bias):
    scores = jnp.dot(x, wt, preferred_element_type=jnp.float32) + bias
    e = scores.shape[-1]
    lane = jax.lax.broadcasted_iota(jnp.int32, scores.shape, 1).astype(
        jnp.float32)

    m1 = jnp.max(scores, axis=-1, keepdims=True)
    idx1 = jnp.min(jnp.where(scores == m1, lane, float(e)), axis=-1,
                   keepdims=True)
    first = lane == idx1
    s2 = jnp.where(first, -jnp.inf, scores)
    m2 = jnp.max(s2, axis=-1, keepdims=True)
    idx2 = jnp.min(jnp.where(s2 == m2, lane, float(e)), axis=-1,
                   keepdims=True)
    mask = first | (lane == idx2)

    p = jnp.exp(scores - m1)
    r = 1.0 / (1.0 + jnp.exp(m2 - m1))
    return jnp.where(mask, p * r, jnp.float32(0.0))


def _gate_kernel(*refs):
    x_refs = refs[:_N_SPLIT]
    wt_ref, b_ref, o_ref = refs[_N_SPLIT:]
    wt = wt_ref[...]
    bias = b_ref[...]
    for j in range(_N_SPLIT):
        w = _gate_rows(x_refs[j][...], wt, bias)
        o_ref[j * _SUB_T:(j + 1) * _SUB_T, :] = w


@jax.jit
def kernel(x, W, b):
    n_tokens, d_model = x.shape
    n_experts = W.shape[0]
    wt = W.T
    b2 = b.reshape(1, n_experts)
    grid = (n_tokens // _BLOCK_T,)
    x_specs = [
        pl.BlockSpec((_SUB_T, d_model),
                     functools.partial(lambda i, j: (_N_SPLIT * i + j, 0), j=j))
        for j in range(_N_SPLIT)
    ]
    return pl.pallas_call(
        _gate_kernel,
        grid=grid,
        in_specs=x_specs + [
            pl.BlockSpec((d_model, n_experts), lambda i: (0, 0)),
            pl.BlockSpec((1, n_experts), lambda i: (0, 0)),
        ],
        out_specs=pl.BlockSpec((_BLOCK_T, n_experts), lambda i: (i, 0)),
        out_shape=jax.ShapeDtypeStruct((n_tokens, n_experts), jnp.float32),
        compiler_params=pltpu.CompilerParams(
            dimension_semantics=("arbitrary",),
        ),
    )(*([x] * _N_SPLIT), wt, b2)
